# initial kernel scaffold (unmeasured)
import jax
import jax.numpy as jnp
from jax import lax
from jax.experimental import pallas as pl
from jax.experimental.pallas import tpu as pltpu


def kernel(x, W):
    T, D = x.shape
    _, Vloc = W.shape
    V = 2 * Vloc

    ROWS = 128

    def body(x_ref, w_ref, out_ref, send_sem, recv_sem):
        my_x = lax.axis_index("x")
        my_y = lax.axis_index("y")
        my_z = lax.axis_index("z")
        peer = (my_x, 1 - my_y, my_z)

        col0 = my_y * Vloc

        for r0 in range(0, T, ROWS):
            out_ref[pl.ds(r0, ROWS), pl.ds(col0, Vloc)] = jnp.dot(
                x_ref[pl.ds(r0, ROWS), :], w_ref[:, :],
                preferred_element_type=jnp.float32,
            )

        barrier_sem = pltpu.get_barrier_semaphore()
        pl.semaphore_signal(
            barrier_sem, inc=1,
            device_id=peer, device_id_type=pl.DeviceIdType.MESH,
        )
        pl.semaphore_wait(barrier_sem, 1)

        rdma = pltpu.make_async_remote_copy(
            src_ref=out_ref.at[:, pl.ds(col0, Vloc)],
            dst_ref=out_ref.at[:, pl.ds(col0, Vloc)],
            send_sem=send_sem,
            recv_sem=recv_sem,
            device_id=peer,
            device_id_type=pl.DeviceIdType.MESH,
        )
        rdma.start()
        rdma.wait()

        for r0 in range(0, T, ROWS):
            blk = out_ref[pl.ds(r0, ROWS), :]
            m = jnp.max(blk, axis=-1, keepdims=True)
            e = jnp.exp(blk - m)
            s = jnp.sum(e, axis=-1, keepdims=True)
            out_ref[pl.ds(r0, ROWS), :] = e / s

    return pl.pallas_call(
        body,
        out_shape=jax.ShapeDtypeStruct((T, V), jnp.float32),
        in_specs=[
            pl.BlockSpec(memory_space=pltpu.VMEM),
            pl.BlockSpec(memory_space=pltpu.VMEM),
        ],
        out_specs=pl.BlockSpec(memory_space=pltpu.VMEM),
        scratch_shapes=[
            pltpu.SemaphoreType.DMA,
            pltpu.SemaphoreType.DMA,
        ],
        compiler_params=pltpu.CompilerParams(
            collective_id=0,
            vmem_limit_bytes=128 * 1024 * 1024,
        ),
    )(x, W)


# baseline (device time: 242390 ns/iter reference)
import jax
import jax.numpy as jnp
from jax import lax
from jax.experimental import pallas as pl
from jax.experimental.pallas import tpu as pltpu

N_CHUNKS = 8


def kernel(x, W):
    T, D = x.shape
    _, Vloc = W.shape
    V = 2 * Vloc
    R = T // N_CHUNKS

    def body(x_ref, w_ref, out_ref, p_buf, stats_send, stats_recv,
             stats_send_sems, stats_recv_sems,
             data_send_sems, data_recv_sems, copy_sems):
        my_x = lax.axis_index("x")
        my_y = lax.axis_index("y")
        my_z = lax.axis_index("z")
        peer = (my_x, 1 - my_y, my_z)
        col0 = my_y * Vloc

        barrier_sem = pltpu.get_barrier_semaphore()
        pl.semaphore_signal(
            barrier_sem, inc=1,
            device_id=peer, device_id_type=pl.DeviceIdType.MESH,
        )
        pl.semaphore_wait(barrier_sem, 1)

        def stats_rdma(r):
            return pltpu.make_async_remote_copy(
                src_ref=stats_send.at[r],
                dst_ref=stats_recv.at[r],
                send_sem=stats_send_sems.at[r],
                recv_sem=stats_recv_sems.at[r],
                device_id=peer,
                device_id_type=pl.DeviceIdType.MESH,
            )

        def data_rdma(r):
            return pltpu.make_async_remote_copy(
                src_ref=p_buf.at[r],
                dst_ref=out_ref.at[pl.ds(r * R, R), pl.ds(col0, Vloc)],
                send_sem=data_send_sems.at[r],
                recv_sem=data_recv_sems.at[r],
                device_id=peer,
                device_id_type=pl.DeviceIdType.MESH,
            )

        def local_copy(r):
            return pltpu.make_async_copy(
                p_buf.at[r],
                out_ref.at[pl.ds(r * R, R), pl.ds(col0, Vloc)],
                copy_sems.at[r],
            )

        for r in range(N_CHUNKS):
            l = jnp.dot(
                x_ref[pl.ds(r * R, R), :], w_ref[:, :],
                preferred_element_type=jnp.float32,
            )
            m_loc = jnp.max(l, axis=-1, keepdims=True)
            e = jnp.exp(l - m_loc)
            s_loc = jnp.sum(e, axis=-1, keepdims=True)
            p_buf[r] = e

            stats_send[r] = jnp.concatenate(
                [jnp.broadcast_to(m_loc, (R, 128)),
                 jnp.broadcast_to(s_loc, (R, 128))], axis=1)
            srd = stats_rdma(r)
            srd.start()
            srd.wait_recv()

            m_peer = stats_recv[r, :, 0:1]
            s_peer = stats_recv[r, :, 128:129]
            m_g = jnp.maximum(m_loc, m_peer)
            s_g = s_loc * jnp.exp(m_loc - m_g) + s_peer * jnp.exp(m_peer - m_g)
            factor = jnp.exp(m_loc - m_g) / s_g

            p_buf[r] = p_buf[r] * factor

            local_copy(r).start()
            data_rdma(r).start()

        for r in range(N_CHUNKS):
            stats_rdma(r).wait_send()
            local_copy(r).wait()
            d = data_rdma(r)
            d.wait_send()
            d.wait_recv()

    return pl.pallas_call(
        body,
        in_specs=[
            pl.BlockSpec(memory_space=pltpu.MemorySpace.VMEM),
            pl.BlockSpec(memory_space=pltpu.MemorySpace.VMEM),
        ],
        out_specs=pl.BlockSpec(memory_space=pl.ANY),
        scratch_shapes=[
            pltpu.VMEM((N_CHUNKS, R, Vloc), jnp.float32),
            pltpu.VMEM((N_CHUNKS, R, 256), jnp.float32),
            pltpu.VMEM((N_CHUNKS, R, 256), jnp.float32),
            pltpu.SemaphoreType.DMA((N_CHUNKS,)),
            pltpu.SemaphoreType.DMA((N_CHUNKS,)),
            pltpu.SemaphoreType.DMA((N_CHUNKS,)),
            pltpu.SemaphoreType.DMA((N_CHUNKS,)),
            pltpu.SemaphoreType.DMA((N_CHUNKS,)),
        ],
        out_shape=jax.ShapeDtypeStruct((T, V), jnp.float32),
        compiler_params=pltpu.CompilerParams(
            collective_id=0,
            vmem_limit_bytes=62 * 1024 * 1024,
        ),
    )(x, W)


# device time: 229912 ns/iter; 1.0543x vs baseline; 1.0543x over previous
import jax
import jax.numpy as jnp
from jax import lax
from jax.experimental import pallas as pl
from jax.experimental.pallas import tpu as pltpu

N_CHUNKS = 8


def kernel(x, W):
    T, D = x.shape
    _, Vloc = W.shape
    V = 2 * Vloc
    R = T // N_CHUNKS

    def body(x_ref, w_ref, out_ref, p_buf, stats_send, stats_recv,
             stats_send_sems, stats_recv_sems,
             data_send_sems, data_recv_sems, copy_sems):
        my_x = lax.axis_index("x")
        my_y = lax.axis_index("y")
        my_z = lax.axis_index("z")
        peer = (my_x, 1 - my_y, my_z)
        col0 = my_y * Vloc

        barrier_sem = pltpu.get_barrier_semaphore()
        pl.semaphore_signal(
            barrier_sem, inc=1,
            device_id=peer, device_id_type=pl.DeviceIdType.MESH,
        )
        pl.semaphore_wait(barrier_sem, 1)

        def stats_rdma(r):
            return pltpu.make_async_remote_copy(
                src_ref=stats_send.at[r],
                dst_ref=stats_recv.at[r],
                send_sem=stats_send_sems.at[r],
                recv_sem=stats_recv_sems.at[r],
                device_id=peer,
                device_id_type=pl.DeviceIdType.MESH,
            )

        def data_rdma(r):
            return pltpu.make_async_remote_copy(
                src_ref=p_buf.at[r],
                dst_ref=out_ref.at[pl.ds(r * R, R), pl.ds(col0, Vloc)],
                send_sem=data_send_sems.at[r],
                recv_sem=data_recv_sems.at[r],
                device_id=peer,
                device_id_type=pl.DeviceIdType.MESH,
            )

        def local_copy(r):
            return pltpu.make_async_copy(
                p_buf.at[r],
                out_ref.at[pl.ds(r * R, R), pl.ds(col0, Vloc)],
                copy_sems.at[r],
            )

        def finalize(r):
            stats_rdma(r).wait_recv()
            m_loc = stats_send[r, :, 0:1]
            s_loc = stats_send[r, :, 128:129]
            m_peer = stats_recv[r, :, 0:1]
            s_peer = stats_recv[r, :, 128:129]
            m_g = jnp.maximum(m_loc, m_peer)
            s_g = s_loc * jnp.exp(m_loc - m_g) + s_peer * jnp.exp(m_peer - m_g)
            factor = jnp.exp(m_loc - m_g) / s_g
            p_buf[r] = p_buf[r] * factor
            local_copy(r).start()
            data_rdma(r).start()

        for r in range(N_CHUNKS):
            l = jnp.dot(
                x_ref[pl.ds(r * R, R), :], w_ref[:, :],
                preferred_element_type=jnp.float32,
            )
            m_loc = jnp.max(l, axis=-1, keepdims=True)
            e = jnp.exp(l - m_loc)
            s_loc = jnp.sum(e, axis=-1, keepdims=True)
            p_buf[r] = e

            stats_send[r] = jnp.concatenate(
                [jnp.broadcast_to(m_loc, (R, 128)),
                 jnp.broadcast_to(s_loc, (R, 128))], axis=1)
            stats_rdma(r).start()

            if r > 0:
                finalize(r - 1)
        finalize(N_CHUNKS - 1)

        for r in range(N_CHUNKS):
            stats_rdma(r).wait_send()
            local_copy(r).wait()
            d = data_rdma(r)
            d.wait_send()
            d.wait_recv()

    return pl.pallas_call(
        body,
        in_specs=[
            pl.BlockSpec(memory_space=pltpu.MemorySpace.VMEM),
            pl.BlockSpec(memory_space=pltpu.MemorySpace.VMEM),
        ],
        out_specs=pl.BlockSpec(memory_space=pl.ANY),
        scratch_shapes=[
            pltpu.VMEM((N_CHUNKS, R, Vloc), jnp.float32),
            pltpu.VMEM((N_CHUNKS, R, 256), jnp.float32),
            pltpu.VMEM((N_CHUNKS, R, 256), jnp.float32),
            pltpu.SemaphoreType.DMA((N_CHUNKS,)),
            pltpu.SemaphoreType.DMA((N_CHUNKS,)),
            pltpu.SemaphoreType.DMA((N_CHUNKS,)),
            pltpu.SemaphoreType.DMA((N_CHUNKS,)),
            pltpu.SemaphoreType.DMA((N_CHUNKS,)),
        ],
        out_shape=jax.ShapeDtypeStruct((T, V), jnp.float32),
        compiler_params=pltpu.CompilerParams(
            collective_id=0,
            vmem_limit_bytes=62 * 1024 * 1024,
        ),
    )(x, W)


# device time: 228687 ns/iter; 1.0599x vs baseline; 1.0054x over previous
import jax
import jax.numpy as jnp
from jax import lax
from jax.experimental import pallas as pl
from jax.experimental.pallas import tpu as pltpu

N_CHUNKS = 4


def kernel(x, W):
    T, D = x.shape
    _, Vloc = W.shape
    V = 2 * Vloc
    R = T // N_CHUNKS

    def body(x_ref, w_ref, out_ref, p_buf, stats_send, stats_recv,
             stats_send_sems, stats_recv_sems,
             data_send_sems, data_recv_sems, copy_sems):
        my_x = lax.axis_index("x")
        my_y = lax.axis_index("y")
        my_z = lax.axis_index("z")
        peer = (my_x, 1 - my_y, my_z)
        col0 = my_y * Vloc

        barrier_sem = pltpu.get_barrier_semaphore()
        pl.semaphore_signal(
            barrier_sem, inc=1,
            device_id=peer, device_id_type=pl.DeviceIdType.MESH,
        )
        pl.semaphore_wait(barrier_sem, 1)

        def stats_rdma(r):
            return pltpu.make_async_remote_copy(
                src_ref=stats_send.at[r],
                dst_ref=stats_recv.at[r],
                send_sem=stats_send_sems.at[r],
                recv_sem=stats_recv_sems.at[r],
                device_id=peer,
                device_id_type=pl.DeviceIdType.MESH,
            )

        def data_rdma(r):
            return pltpu.make_async_remote_copy(
                src_ref=p_buf.at[r],
                dst_ref=out_ref.at[pl.ds(r * R, R), pl.ds(col0, Vloc)],
                send_sem=data_send_sems.at[r],
                recv_sem=data_recv_sems.at[r],
                device_id=peer,
                device_id_type=pl.DeviceIdType.MESH,
            )

        def local_copy(r):
            return pltpu.make_async_copy(
                p_buf.at[r],
                out_ref.at[pl.ds(r * R, R), pl.ds(col0, Vloc)],
                copy_sems.at[r],
            )

        def finalize(r):
            stats_rdma(r).wait_recv()
            m_loc = stats_send[r, :, 0:1]
            s_loc = stats_send[r, :, 128:129]
            m_peer = stats_recv[r, :, 0:1]
            s_peer = stats_recv[r, :, 128:129]
            m_g = jnp.maximum(m_loc, m_peer)
            s_g = s_loc * jnp.exp(m_loc - m_g) + s_peer * jnp.exp(m_peer - m_g)
            factor = jnp.exp(m_loc - m_g) / s_g
            p_buf[r] = p_buf[r] * factor
            local_copy(r).start()
            data_rdma(r).start()

        for r in range(N_CHUNKS):
            l = jnp.dot(
                x_ref[pl.ds(r * R, R), :], w_ref[:, :],
                preferred_element_type=jnp.float32,
            )
            m_loc = jnp.max(l, axis=-1, keepdims=True)
            e = jnp.exp(l - m_loc)
            s_loc = jnp.sum(e, axis=-1, keepdims=True)
            p_buf[r] = e

            stats_send[r] = jnp.concatenate(
                [jnp.broadcast_to(m_loc, (R, 128)),
                 jnp.broadcast_to(s_loc, (R, 128))], axis=1)
            stats_rdma(r).start()

            if r > 0:
                finalize(r - 1)
        finalize(N_CHUNKS - 1)

        for r in range(N_CHUNKS):
            stats_rdma(r).wait_send()
            local_copy(r).wait()
            d = data_rdma(r)
            d.wait_send()
            d.wait_recv()

    return pl.pallas_call(
        body,
        in_specs=[
            pl.BlockSpec(memory_space=pltpu.MemorySpace.VMEM),
            pl.BlockSpec(memory_space=pltpu.MemorySpace.VMEM),
        ],
        out_specs=pl.BlockSpec(memory_space=pl.ANY),
        scratch_shapes=[
            pltpu.VMEM((N_CHUNKS, R, Vloc), jnp.float32),
            pltpu.VMEM((N_CHUNKS, R, 256), jnp.float32),
            pltpu.VMEM((N_CHUNKS, R, 256), jnp.float32),
            pltpu.SemaphoreType.DMA((N_CHUNKS,)),
            pltpu.SemaphoreType.DMA((N_CHUNKS,)),
            pltpu.SemaphoreType.DMA((N_CHUNKS,)),
            pltpu.SemaphoreType.DMA((N_CHUNKS,)),
            pltpu.SemaphoreType.DMA((N_CHUNKS,)),
        ],
        out_shape=jax.ShapeDtypeStruct((T, V), jnp.float32),
        compiler_params=pltpu.CompilerParams(
            collective_id=0,
            vmem_limit_bytes=62 * 1024 * 1024,
        ),
    )(x, W)


# device time: 132584 ns/iter; 1.8282x vs baseline; 1.7248x over previous
import jax
import jax.numpy as jnp
from jax import lax
from jax.experimental import pallas as pl
from jax.experimental.pallas import tpu as pltpu

N_CHUNKS = 8


def _gemm_exchange(x, W):
    T, D = x.shape
    _, Vloc = W.shape
    V = 2 * Vloc
    R = T // N_CHUNKS

    def body(x_ref, w_ref, e_all_ref, factors_ref,
             e_buf, stats_send, stats_recv,
             stats_send_sems, stats_recv_sems,
             data_send_sems, data_recv_sems, copy_sems):
        my_x = lax.axis_index("x")
        my_y = lax.axis_index("y")
        my_z = lax.axis_index("z")
        peer = (my_x, 1 - my_y, my_z)
        col0 = my_y * Vloc

        barrier_sem = pltpu.get_barrier_semaphore()
        pl.semaphore_signal(
            barrier_sem, inc=1,
            device_id=peer, device_id_type=pl.DeviceIdType.MESH,
        )
        pl.semaphore_wait(barrier_sem, 1)

        def stats_rdma(r):
            return pltpu.make_async_remote_copy(
                src_ref=stats_send.at[r],
                dst_ref=stats_recv.at[r],
                send_sem=stats_send_sems.at[r],
                recv_sem=stats_recv_sems.at[r],
                device_id=peer,
                device_id_type=pl.DeviceIdType.MESH,
            )

        def data_rdma(r):
            return pltpu.make_async_remote_copy(
                src_ref=e_buf.at[r],
                dst_ref=e_all_ref.at[pl.ds(r * R, R), pl.ds(col0, Vloc)],
                send_sem=data_send_sems.at[r],
                recv_sem=data_recv_sems.at[r],
                device_id=peer,
                device_id_type=pl.DeviceIdType.MESH,
            )

        def local_copy(r):
            return pltpu.make_async_copy(
                e_buf.at[r],
                e_all_ref.at[pl.ds(r * R, R), pl.ds(col0, Vloc)],
                copy_sems.at[r],
            )

        for r in range(N_CHUNKS):
            l = jnp.dot(
                x_ref[pl.ds(r * R, R), :], w_ref[:, :],
                preferred_element_type=jnp.float32,
            )
            m_loc = jnp.max(l, axis=-1, keepdims=True)
            e = jnp.exp(l - m_loc)
            s_loc = jnp.sum(e, axis=-1, keepdims=True)
            e_buf[r] = e.astype(jnp.bfloat16)

            stats_send[r] = jnp.concatenate(
                [jnp.broadcast_to(m_loc, (R, 128)),
                 jnp.broadcast_to(s_loc, (R, 128))], axis=1)
            stats_rdma(r).start()
            local_copy(r).start()
            data_rdma(r).start()

        for r in range(N_CHUNKS):
            stats_rdma(r).wait_recv()
            m_loc = stats_send[r, :, 0:1]
            s_loc = stats_send[r, :, 128:129]
            m_peer = stats_recv[r, :, 0:1]
            s_peer = stats_recv[r, :, 128:129]
            m_g = jnp.maximum(m_loc, m_peer)
            s_g = s_loc * jnp.exp(m_loc - m_g) + s_peer * jnp.exp(m_peer - m_g)
            f_mine = jnp.exp(m_loc - m_g) / s_g
            f_peer = jnp.exp(m_peer - m_g) / s_g
            half0 = jnp.where(my_y == 0, f_mine, f_peer)
            half1 = jnp.where(my_y == 0, f_peer, f_mine)
            factors_ref[pl.ds(r * R, R), :] = jnp.concatenate(
                [jnp.broadcast_to(half0, (R, 128)),
                 jnp.broadcast_to(half1, (R, 128))], axis=1)

        for r in range(N_CHUNKS):
            stats_rdma(r).wait_send()
            local_copy(r).wait()
            d = data_rdma(r)
            d.wait_send()
            d.wait_recv()

    return pl.pallas_call(
        body,
        in_specs=[
            pl.BlockSpec(memory_space=pltpu.MemorySpace.VMEM),
            pl.BlockSpec(memory_space=pltpu.MemorySpace.VMEM),
        ],
        out_specs=(
            pl.BlockSpec(memory_space=pl.ANY),
            pl.BlockSpec(memory_space=pltpu.MemorySpace.VMEM),
        ),
        out_shape=(
            jax.ShapeDtypeStruct((T, V), jnp.bfloat16),
            jax.ShapeDtypeStruct((T, 256), jnp.float32),
        ),
        scratch_shapes=[
            pltpu.VMEM((N_CHUNKS, R, Vloc), jnp.bfloat16),
            pltpu.VMEM((N_CHUNKS, R, 256), jnp.float32),
            pltpu.VMEM((N_CHUNKS, R, 256), jnp.float32),
            pltpu.SemaphoreType.DMA((N_CHUNKS,)),
            pltpu.SemaphoreType.DMA((N_CHUNKS,)),
            pltpu.SemaphoreType.DMA((N_CHUNKS,)),
            pltpu.SemaphoreType.DMA((N_CHUNKS,)),
            pltpu.SemaphoreType.DMA((N_CHUNKS,)),
        ],
        compiler_params=pltpu.CompilerParams(
            collective_id=0,
            vmem_limit_bytes=62 * 1024 * 1024,
        ),
    )(x, W)


def _normalize(e_all, factors):
    T, V = e_all.shape
    H = V // 2
    ROWS = 128

    def body(e_ref, f_ref, o_ref):
        f0 = f_ref[:, 0:1]
        f1 = f_ref[:, 128:129]
        o_ref[:, :H] = e_ref[:, :H].astype(jnp.float32) * f0
        o_ref[:, H:] = e_ref[:, H:].astype(jnp.float32) * f1

    return pl.pallas_call(
        body,
        grid=(T // ROWS,),
        in_specs=[
            pl.BlockSpec((ROWS, V), lambda i: (i, 0)),
            pl.BlockSpec((ROWS, 256), lambda i: (i, 0)),
        ],
        out_specs=pl.BlockSpec((ROWS, V), lambda i: (i, 0)),
        out_shape=jax.ShapeDtypeStruct((T, V), jnp.float32),
        compiler_params=pltpu.CompilerParams(
            vmem_limit_bytes=62 * 1024 * 1024,
        ),
    )(e_all, factors)


def kernel(x, W):
    e_all, factors = _gemm_exchange(x, W)
    return _normalize(e_all, factors)


# device time: 104752 ns/iter; 2.3139x vs baseline; 1.2657x over previous
import jax
import jax.numpy as jnp
from jax import lax
from jax.experimental import pallas as pl
from jax.experimental.pallas import tpu as pltpu

N_CHUNKS = 8
N_HALF = N_CHUNKS // 2


def _gemm_exchange(x, W):
    T, D = x.shape
    _, Vloc = W.shape
    V = 2 * Vloc
    R = T // N_CHUNKS

    def body(x_ref, w_ref, e_all_ref, factors_ref,
             e_buf, stats_send, stats_recv,
             stats_send_sems, stats_recv_sems,
             y_send_sems, y_recv_sems, fwd_send_sems, fwd_recv_sems,
             copy_sems):
        my_x = lax.axis_index("x")
        my_y = lax.axis_index("y")
        my_z = lax.axis_index("z")
        y_peer = (my_x, 1 - my_y, my_z)
        x_peer = (1 - my_x, my_y, my_z)
        col0 = my_y * Vloc
        pcol0 = (1 - my_y) * Vloc

        barrier_sem = pltpu.get_barrier_semaphore()
        for nbr in (y_peer, x_peer):
            pl.semaphore_signal(
                barrier_sem, inc=1,
                device_id=nbr, device_id_type=pl.DeviceIdType.MESH,
            )
        pl.semaphore_wait(barrier_sem, 2)

        def stats_rdma(r):
            return pltpu.make_async_remote_copy(
                src_ref=stats_send.at[r],
                dst_ref=stats_recv.at[r],
                send_sem=stats_send_sems.at[r],
                recv_sem=stats_recv_sems.at[r],
                device_id=y_peer,
                device_id_type=pl.DeviceIdType.MESH,
            )

        def y_rdma(k):
            idx = 2 * k + my_x
            return pltpu.make_async_remote_copy(
                src_ref=e_buf.at[idx],
                dst_ref=e_all_ref.at[pl.ds(idx * R, R), pl.ds(col0, Vloc)],
                send_sem=y_send_sems.at[k],
                recv_sem=y_recv_sems.at[k],
                device_id=y_peer,
                device_id_type=pl.DeviceIdType.MESH,
            )

        def fwd_rdma(k):
            idx = 2 * k + my_x
            region = (pl.ds(idx * R, R), pl.ds(pcol0, Vloc))
            return pltpu.make_async_remote_copy(
                src_ref=e_all_ref.at[region[0], region[1]],
                dst_ref=e_all_ref.at[region[0], region[1]],
                send_sem=fwd_send_sems.at[k],
                recv_sem=fwd_recv_sems.at[k],
                device_id=x_peer,
                device_id_type=pl.DeviceIdType.MESH,
            )

        def local_copy(r):
            return pltpu.make_async_copy(
                e_buf.at[r],
                e_all_ref.at[pl.ds(r * R, R), pl.ds(col0, Vloc)],
                copy_sems.at[r],
            )

        def forward(k):
            y_rdma(k).wait_recv()
            fwd_rdma(k).start()

        for k in range(N_HALF):
            for r in (2 * k, 2 * k + 1):
                l = jnp.dot(
                    x_ref[pl.ds(r * R, R), :], w_ref[:, :],
                    preferred_element_type=jnp.float32,
                )
                m_loc = jnp.max(l, axis=-1, keepdims=True)
                e = jnp.exp(l - m_loc)
                s_loc = jnp.sum(e, axis=-1, keepdims=True)
                e_buf[r] = e.astype(jnp.bfloat16)

                stats_send[r] = jnp.concatenate(
                    [jnp.broadcast_to(m_loc, (R, 128)),
                     jnp.broadcast_to(s_loc, (R, 128))], axis=1)
                stats_rdma(r).start()
                local_copy(r).start()

            y_rdma(k).start()
            if k > 0:
                forward(k - 1)
        forward(N_HALF - 1)

        for r in range(N_CHUNKS):
            stats_rdma(r).wait_recv()
            m_loc = stats_send[r, :, 0:1]
            s_loc = stats_send[r, :, 128:129]
            m_peer = stats_recv[r, :, 0:1]
            s_peer = stats_recv[r, :, 128:129]
            m_g = jnp.maximum(m_loc, m_peer)
            s_g = s_loc * jnp.exp(m_loc - m_g) + s_peer * jnp.exp(m_peer - m_g)
            f_mine = jnp.exp(m_loc - m_g) / s_g
            f_peer = jnp.exp(m_peer - m_g) / s_g
            half0 = jnp.where(my_y == 0, f_mine, f_peer)
            half1 = jnp.where(my_y == 0, f_peer, f_mine)
            factors_ref[pl.ds(r * R, R), :] = jnp.concatenate(
                [jnp.broadcast_to(half0, (R, 128)),
                 jnp.broadcast_to(half1, (R, 128))], axis=1)

        for r in range(N_CHUNKS):
            stats_rdma(r).wait_send()
            local_copy(r).wait()
        for k in range(N_HALF):
            y_rdma(k).wait_send()
            f = fwd_rdma(k)
            f.wait_send()
            f.wait_recv()

    return pl.pallas_call(
        body,
        in_specs=[
            pl.BlockSpec(memory_space=pltpu.MemorySpace.VMEM),
            pl.BlockSpec(memory_space=pltpu.MemorySpace.VMEM),
        ],
        out_specs=(
            pl.BlockSpec(memory_space=pl.ANY),
            pl.BlockSpec(memory_space=pltpu.MemorySpace.VMEM),
        ),
        out_shape=(
            jax.ShapeDtypeStruct((T, V), jnp.bfloat16),
            jax.ShapeDtypeStruct((T, 256), jnp.float32),
        ),
        scratch_shapes=[
            pltpu.VMEM((N_CHUNKS, R, Vloc), jnp.bfloat16),
            pltpu.VMEM((N_CHUNKS, R, 256), jnp.float32),
            pltpu.VMEM((N_CHUNKS, R, 256), jnp.float32),
            pltpu.SemaphoreType.DMA((N_CHUNKS,)),
            pltpu.SemaphoreType.DMA((N_CHUNKS,)),
            pltpu.SemaphoreType.DMA((N_HALF,)),
            pltpu.SemaphoreType.DMA((N_HALF,)),
            pltpu.SemaphoreType.DMA((N_HALF,)),
            pltpu.SemaphoreType.DMA((N_HALF,)),
            pltpu.SemaphoreType.DMA((N_CHUNKS,)),
        ],
        compiler_params=pltpu.CompilerParams(
            collective_id=0,
            vmem_limit_bytes=62 * 1024 * 1024,
        ),
    )(x, W)


def _normalize(e_all, factors):
    T, V = e_all.shape
    H = V // 2
    ROWS = 128

    def body(e_ref, f_ref, o_ref):
        f0 = f_ref[:, 0:1]
        f1 = f_ref[:, 128:129]
        o_ref[:, :H] = e_ref[:, :H].astype(jnp.float32) * f0
        o_ref[:, H:] = e_ref[:, H:].astype(jnp.float32) * f1

    return pl.pallas_call(
        body,
        grid=(T // ROWS,),
        in_specs=[
            pl.BlockSpec((ROWS, V), lambda i: (i, 0)),
            pl.BlockSpec((ROWS, 256), lambda i: (i, 0)),
        ],
        out_specs=pl.BlockSpec((ROWS, V), lambda i: (i, 0)),
        out_shape=jax.ShapeDtypeStruct((T, V), jnp.float32),
        compiler_params=pltpu.CompilerParams(
            vmem_limit_bytes=62 * 1024 * 1024,
        ),
    )(e_all, factors)


def kernel(x, W):
    e_all, factors = _gemm_exchange(x, W)
    return _normalize(e_all, factors)


# device time: 104704 ns/iter; 2.3150x vs baseline; 1.0005x over previous
import jax
import jax.numpy as jnp
from jax import lax
from jax.experimental import pallas as pl
from jax.experimental.pallas import tpu as pltpu

N_CHUNKS = 16
N_HALF = N_CHUNKS // 2


def _gemm_exchange(x, W):
    T, D = x.shape
    _, Vloc = W.shape
    V = 2 * Vloc
    R = T // N_CHUNKS

    def body(x_ref, w_ref, e_all_ref, factors_ref,
             e_buf, stats_send, stats_recv,
             stats_send_sems, stats_recv_sems,
             y_send_sems, y_recv_sems, fwd_send_sems, fwd_recv_sems,
             copy_sems):
        my_x = lax.axis_index("x")
        my_y = lax.axis_index("y")
        my_z = lax.axis_index("z")
        y_peer = (my_x, 1 - my_y, my_z)
        x_peer = (1 - my_x, my_y, my_z)
        col0 = my_y * Vloc
        pcol0 = (1 - my_y) * Vloc

        barrier_sem = pltpu.get_barrier_semaphore()
        for nbr in (y_peer, x_peer):
            pl.semaphore_signal(
                barrier_sem, inc=1,
                device_id=nbr, device_id_type=pl.DeviceIdType.MESH,
            )
        pl.semaphore_wait(barrier_sem, 2)

        def stats_rdma(r):
            return pltpu.make_async_remote_copy(
                src_ref=stats_send.at[r],
                dst_ref=stats_recv.at[r],
                send_sem=stats_send_sems.at[r],
                recv_sem=stats_recv_sems.at[r],
                device_id=y_peer,
                device_id_type=pl.DeviceIdType.MESH,
            )

        def y_rdma(k):
            idx = 2 * k + my_x
            return pltpu.make_async_remote_copy(
                src_ref=e_buf.at[idx],
                dst_ref=e_all_ref.at[pl.ds(idx * R, R), pl.ds(col0, Vloc)],
                send_sem=y_send_sems.at[k],
                recv_sem=y_recv_sems.at[k],
                device_id=y_peer,
                device_id_type=pl.DeviceIdType.MESH,
            )

        def fwd_rdma(k):
            idx = 2 * k + my_x
            region = (pl.ds(idx * R, R), pl.ds(pcol0, Vloc))
            return pltpu.make_async_remote_copy(
                src_ref=e_all_ref.at[region[0], region[1]],
                dst_ref=e_all_ref.at[region[0], region[1]],
                send_sem=fwd_send_sems.at[k],
                recv_sem=fwd_recv_sems.at[k],
                device_id=x_peer,
                device_id_type=pl.DeviceIdType.MESH,
            )

        def local_copy(r):
            return pltpu.make_async_copy(
                e_buf.at[r],
                e_all_ref.at[pl.ds(r * R, R), pl.ds(col0, Vloc)],
                copy_sems.at[r],
            )

        def forward(k):
            y_rdma(k).wait_recv()
            fwd_rdma(k).start()

        for k in range(N_HALF):
            for r in (2 * k, 2 * k + 1):
                l = jnp.dot(
                    x_ref[pl.ds(r * R, R), :], w_ref[:, :],
                    preferred_element_type=jnp.float32,
                )
                m_loc = jnp.max(l, axis=-1, keepdims=True)
                e = jnp.exp(l - m_loc)
                s_loc = jnp.sum(e, axis=-1, keepdims=True)
                e_buf[r] = e.astype(jnp.bfloat16)

                stats_send[r] = jnp.concatenate(
                    [jnp.broadcast_to(m_loc, (R, 128)),
                     jnp.broadcast_to(s_loc, (R, 128))], axis=1)
                stats_rdma(r).start()
                local_copy(r).start()

            y_rdma(k).start()
            if k > 0:
                forward(k - 1)
        forward(N_HALF - 1)

        for r in range(N_CHUNKS):
            stats_rdma(r).wait_recv()
            m_loc = stats_send[r, :, 0:1]
            s_loc = stats_send[r, :, 128:129]
            m_peer = stats_recv[r, :, 0:1]
            s_peer = stats_recv[r, :, 128:129]
            m_g = jnp.maximum(m_loc, m_peer)
            s_g = s_loc * jnp.exp(m_loc - m_g) + s_peer * jnp.exp(m_peer - m_g)
            f_mine = jnp.exp(m_loc - m_g) / s_g
            f_peer = jnp.exp(m_peer - m_g) / s_g
            half0 = jnp.where(my_y == 0, f_mine, f_peer)
            half1 = jnp.where(my_y == 0, f_peer, f_mine)
            factors_ref[pl.ds(r * R, R), :] = jnp.concatenate(
                [jnp.broadcast_to(half0, (R, 128)),
                 jnp.broadcast_to(half1, (R, 128))], axis=1)

        for r in range(N_CHUNKS):
            stats_rdma(r).wait_send()
            local_copy(r).wait()
        for k in range(N_HALF):
            y_rdma(k).wait_send()
            f = fwd_rdma(k)
            f.wait_send()
            f.wait_recv()

    return pl.pallas_call(
        body,
        in_specs=[
            pl.BlockSpec(memory_space=pltpu.MemorySpace.VMEM),
            pl.BlockSpec(memory_space=pltpu.MemorySpace.VMEM),
        ],
        out_specs=(
            pl.BlockSpec(memory_space=pl.ANY),
            pl.BlockSpec(memory_space=pltpu.MemorySpace.VMEM),
        ),
        out_shape=(
            jax.ShapeDtypeStruct((T, V), jnp.bfloat16),
            jax.ShapeDtypeStruct((T, 256), jnp.float32),
        ),
        scratch_shapes=[
            pltpu.VMEM((N_CHUNKS, R, Vloc), jnp.bfloat16),
            pltpu.VMEM((N_CHUNKS, R, 256), jnp.float32),
            pltpu.VMEM((N_CHUNKS, R, 256), jnp.float32),
            pltpu.SemaphoreType.DMA((N_CHUNKS,)),
            pltpu.SemaphoreType.DMA((N_CHUNKS,)),
            pltpu.SemaphoreType.DMA((N_HALF,)),
            pltpu.SemaphoreType.DMA((N_HALF,)),
            pltpu.SemaphoreType.DMA((N_HALF,)),
            pltpu.SemaphoreType.DMA((N_HALF,)),
            pltpu.SemaphoreType.DMA((N_CHUNKS,)),
        ],
        compiler_params=pltpu.CompilerParams(
            collective_id=0,
            vmem_limit_bytes=62 * 1024 * 1024,
        ),
    )(x, W)


def _normalize(e_all, factors):
    T, V = e_all.shape
    H = V // 2
    ROWS = 128

    def body(e_ref, f_ref, o_ref):
        f0 = f_ref[:, 0:1]
        f1 = f_ref[:, 128:129]
        o_ref[:, :H] = e_ref[:, :H].astype(jnp.float32) * f0
        o_ref[:, H:] = e_ref[:, H:].astype(jnp.float32) * f1

    return pl.pallas_call(
        body,
        grid=(T // ROWS,),
        in_specs=[
            pl.BlockSpec((ROWS, V), lambda i: (i, 0)),
            pl.BlockSpec((ROWS, 256), lambda i: (i, 0)),
        ],
        out_specs=pl.BlockSpec((ROWS, V), lambda i: (i, 0)),
        out_shape=jax.ShapeDtypeStruct((T, V), jnp.float32),
        compiler_params=pltpu.CompilerParams(
            vmem_limit_bytes=62 * 1024 * 1024,
        ),
    )(e_all, factors)


def kernel(x, W):
    e_all, factors = _gemm_exchange(x, W)
    return _normalize(e_all, factors)


# device time: 103973 ns/iter; 2.3313x vs baseline; 1.0070x over previous
import jax
import jax.numpy as jnp
from jax import lax
from jax.experimental import pallas as pl
from jax.experimental.pallas import tpu as pltpu

N_CHUNKS = 8
N_HALF = N_CHUNKS // 2
N_GBLK = 2


def _gemm_exchange(x, W):
    T, D = x.shape
    _, Vloc = W.shape
    V = 2 * Vloc
    R = T // N_CHUNKS

    def body(x_ref, w_ref, e_all_ref, factors_ref,
             e_buf, stats_send, stats_recv,
             stats_send_sems, stats_recv_sems,
             y_send_sems, y_recv_sems, fwd_send_sems, fwd_recv_sems,
             copy_sems):
        my_x = lax.axis_index("x")
        my_y = lax.axis_index("y")
        my_z = lax.axis_index("z")
        y_peer = (my_x, 1 - my_y, my_z)
        x_peer = (1 - my_x, my_y, my_z)
        col0 = my_y * Vloc
        pcol0 = (1 - my_y) * Vloc

        barrier_sem = pltpu.get_barrier_semaphore()
        for nbr in (y_peer, x_peer):
            pl.semaphore_signal(
                barrier_sem, inc=1,
                device_id=nbr, device_id_type=pl.DeviceIdType.MESH,
            )
        pl.semaphore_wait(barrier_sem, 2)

        def stats_rdma(r):
            return pltpu.make_async_remote_copy(
                src_ref=stats_send.at[r],
                dst_ref=stats_recv.at[r],
                send_sem=stats_send_sems.at[r],
                recv_sem=stats_recv_sems.at[r],
                device_id=y_peer,
                device_id_type=pl.DeviceIdType.MESH,
            )

        def y_rdma(k):
            idx = 2 * k + my_x
            return pltpu.make_async_remote_copy(
                src_ref=e_buf.at[idx],
                dst_ref=e_all_ref.at[pl.ds(idx * R, R), pl.ds(col0, Vloc)],
                send_sem=y_send_sems.at[k],
                recv_sem=y_recv_sems.at[k],
                device_id=y_peer,
                device_id_type=pl.DeviceIdType.MESH,
            )

        def fwd_rdma(k):
            idx = 2 * k + my_x
            region = (pl.ds(idx * R, R), pl.ds(pcol0, Vloc))
            return pltpu.make_async_remote_copy(
                src_ref=e_all_ref.at[region[0], region[1]],
                dst_ref=e_all_ref.at[region[0], region[1]],
                send_sem=fwd_send_sems.at[k],
                recv_sem=fwd_recv_sems.at[k],
                device_id=x_peer,
                device_id_type=pl.DeviceIdType.MESH,
            )

        def local_copy(r):
            return pltpu.make_async_copy(
                e_buf.at[r],
                e_all_ref.at[pl.ds(r * R, R), pl.ds(col0, Vloc)],
                copy_sems.at[r],
            )

        def forward(k):
            y_rdma(k).wait_recv()
            fwd_rdma(k).start()

        CPG = N_CHUNKS // N_GBLK
        GR = T // N_GBLK
        for g in range(N_GBLK):
            l = jnp.dot(
                x_ref[pl.ds(g * GR, GR), :], w_ref[:, :],
                preferred_element_type=jnp.float32,
            )
            m_loc = jnp.max(l, axis=-1, keepdims=True)
            e = jnp.exp(l - m_loc).astype(jnp.bfloat16)
            s_loc = jnp.sum(e.astype(jnp.float32), axis=-1, keepdims=True)
            for j in range(CPG):
                r = g * CPG + j
                e_buf[r] = e[j * R:(j + 1) * R, :]
                stats_send[r] = jnp.concatenate(
                    [jnp.broadcast_to(m_loc[j * R:(j + 1) * R], (R, 128)),
                     jnp.broadcast_to(s_loc[j * R:(j + 1) * R], (R, 128))],
                    axis=1)
                stats_rdma(r).start()
                local_copy(r).start()
            for k in range(g * CPG // 2, (g + 1) * CPG // 2):
                y_rdma(k).start()
            if g > 0:
                for k in range((g - 1) * CPG // 2, g * CPG // 2):
                    forward(k)
        for k in range((N_GBLK - 1) * CPG // 2, N_HALF):
            forward(k)

        for r in range(N_CHUNKS):
            stats_rdma(r).wait_recv()
            m_loc = stats_send[r, :, 0:1]
            s_loc = stats_send[r, :, 128:129]
            m_peer = stats_recv[r, :, 0:1]
            s_peer = stats_recv[r, :, 128:129]
            m_g = jnp.maximum(m_loc, m_peer)
            s_g = s_loc * jnp.exp(m_loc - m_g) + s_peer * jnp.exp(m_peer - m_g)
            f_mine = jnp.exp(m_loc - m_g) / s_g
            f_peer = jnp.exp(m_peer - m_g) / s_g
            half0 = jnp.where(my_y == 0, f_mine, f_peer)
            half1 = jnp.where(my_y == 0, f_peer, f_mine)
            factors_ref[pl.ds(r * R, R), :] = jnp.concatenate(
                [jnp.broadcast_to(half0, (R, 128)),
                 jnp.broadcast_to(half1, (R, 128))], axis=1)

        for r in range(N_CHUNKS):
            stats_rdma(r).wait_send()
            local_copy(r).wait()
        for k in range(N_HALF):
            y_rdma(k).wait_send()
            f = fwd_rdma(k)
            f.wait_send()
            f.wait_recv()

    return pl.pallas_call(
        body,
        in_specs=[
            pl.BlockSpec(memory_space=pltpu.MemorySpace.VMEM),
            pl.BlockSpec(memory_space=pltpu.MemorySpace.VMEM),
        ],
        out_specs=(
            pl.BlockSpec(memory_space=pl.ANY),
            pl.BlockSpec(memory_space=pltpu.MemorySpace.VMEM),
        ),
        out_shape=(
            jax.ShapeDtypeStruct((T, V), jnp.bfloat16),
            jax.ShapeDtypeStruct((T, 256), jnp.float32),
        ),
        scratch_shapes=[
            pltpu.VMEM((N_CHUNKS, R, Vloc), jnp.bfloat16),
            pltpu.VMEM((N_CHUNKS, R, 256), jnp.float32),
            pltpu.VMEM((N_CHUNKS, R, 256), jnp.float32),
            pltpu.SemaphoreType.DMA((N_CHUNKS,)),
            pltpu.SemaphoreType.DMA((N_CHUNKS,)),
            pltpu.SemaphoreType.DMA((N_HALF,)),
            pltpu.SemaphoreType.DMA((N_HALF,)),
            pltpu.SemaphoreType.DMA((N_HALF,)),
            pltpu.SemaphoreType.DMA((N_HALF,)),
            pltpu.SemaphoreType.DMA((N_CHUNKS,)),
        ],
        compiler_params=pltpu.CompilerParams(
            collective_id=0,
            vmem_limit_bytes=62 * 1024 * 1024,
        ),
    )(x, W)


def _normalize(e_all, factors):
    T, V = e_all.shape
    H = V // 2
    ROWS = 128

    def body(e_ref, f_ref, o_ref):
        f0 = f_ref[:, 0:1]
        f1 = f_ref[:, 128:129]
        o_ref[:, :H] = e_ref[:, :H].astype(jnp.float32) * f0
        o_ref[:, H:] = e_ref[:, H:].astype(jnp.float32) * f1

    return pl.pallas_call(
        body,
        grid=(T // ROWS,),
        in_specs=[
            pl.BlockSpec((ROWS, V), lambda i: (i, 0)),
            pl.BlockSpec((ROWS, 256), lambda i: (i, 0)),
        ],
        out_specs=pl.BlockSpec((ROWS, V), lambda i: (i, 0)),
        out_shape=jax.ShapeDtypeStruct((T, V), jnp.float32),
        compiler_params=pltpu.CompilerParams(
            vmem_limit_bytes=62 * 1024 * 1024,
        ),
    )(e_all, factors)


def kernel(x, W):
    e_all, factors = _gemm_exchange(x, W)
    return _normalize(e_all, factors)


# device time: 95749 ns/iter; 2.5315x vs baseline; 1.0859x over previous
import jax
import jax.numpy as jnp
from jax import lax
from jax.experimental import pallas as pl
from jax.experimental.pallas import tpu as pltpu

N_CHUNKS = 16
N_HALF = N_CHUNKS // 2
N_GBLK = 4


def _gemm_exchange(x, W):
    T, D = x.shape
    _, Vloc = W.shape
    V = 2 * Vloc
    R = T // N_CHUNKS

    def body(x_ref, w_ref, e_all_ref, factors_ref,
             e_buf, stats_send, stats_recv,
             stats_send_sems, stats_recv_sems,
             y_send_sems, y_recv_sems, fwd_send_sems, fwd_recv_sems,
             copy_sems):
        my_x = lax.axis_index("x")
        my_y = lax.axis_index("y")
        my_z = lax.axis_index("z")
        y_peer = (my_x, 1 - my_y, my_z)
        x_peer = (1 - my_x, my_y, my_z)
        col0 = my_y * Vloc
        pcol0 = (1 - my_y) * Vloc

        barrier_sem = pltpu.get_barrier_semaphore()
        for nbr in (y_peer, x_peer):
            pl.semaphore_signal(
                barrier_sem, inc=1,
                device_id=nbr, device_id_type=pl.DeviceIdType.MESH,
            )
        pl.semaphore_wait(barrier_sem, 2)

        def stats_rdma(r):
            return pltpu.make_async_remote_copy(
                src_ref=stats_send.at[r],
                dst_ref=stats_recv.at[r],
                send_sem=stats_send_sems.at[r],
                recv_sem=stats_recv_sems.at[r],
                device_id=y_peer,
                device_id_type=pl.DeviceIdType.MESH,
            )

        def y_rdma(k):
            idx = 2 * k + my_x
            return pltpu.make_async_remote_copy(
                src_ref=e_buf.at[idx],
                dst_ref=e_all_ref.at[pl.ds(idx * R, R), pl.ds(col0, Vloc)],
                send_sem=y_send_sems.at[k],
                recv_sem=y_recv_sems.at[k],
                device_id=y_peer,
                device_id_type=pl.DeviceIdType.MESH,
            )

        def fwd_rdma(k):
            idx = 2 * k + my_x
            region = (pl.ds(idx * R, R), pl.ds(pcol0, Vloc))
            return pltpu.make_async_remote_copy(
                src_ref=e_all_ref.at[region[0], region[1]],
                dst_ref=e_all_ref.at[region[0], region[1]],
                send_sem=fwd_send_sems.at[k],
                recv_sem=fwd_recv_sems.at[k],
                device_id=x_peer,
                device_id_type=pl.DeviceIdType.MESH,
            )

        def local_copy(r):
            return pltpu.make_async_copy(
                e_buf.at[r],
                e_all_ref.at[pl.ds(r * R, R), pl.ds(col0, Vloc)],
                copy_sems.at[r],
            )

        def forward(k):
            y_rdma(k).wait_recv()
            fwd_rdma(k).start()

        CPG = N_CHUNKS // N_GBLK
        GR = T // N_GBLK
        for g in range(N_GBLK):
            l = jnp.dot(
                x_ref[pl.ds(g * GR, GR), :], w_ref[:, :],
                preferred_element_type=jnp.float32,
            )
            m_loc = jnp.max(l, axis=-1, keepdims=True)
            e = jnp.exp(l - m_loc).astype(jnp.bfloat16)
            s_loc = jnp.sum(e.astype(jnp.float32), axis=-1, keepdims=True)
            for j in range(CPG):
                r = g * CPG + j
                e_buf[r] = e[j * R:(j + 1) * R, :]
                stats_send[r] = jnp.concatenate(
                    [jnp.broadcast_to(m_loc[j * R:(j + 1) * R], (R, 128)),
                     jnp.broadcast_to(s_loc[j * R:(j + 1) * R], (R, 128))],
                    axis=1)
                stats_rdma(r).start()
                local_copy(r).start()
            for k in range(g * CPG // 2, (g + 1) * CPG // 2):
                y_rdma(k).start()
            if g > 0:
                for k in range((g - 1) * CPG // 2, g * CPG // 2):
                    forward(k)
        for k in range((N_GBLK - 1) * CPG // 2, N_HALF):
            forward(k)

        for r in range(N_CHUNKS):
            stats_rdma(r).wait_recv()
            m_loc = stats_send[r, :, 0:1]
            s_loc = stats_send[r, :, 128:129]
            m_peer = stats_recv[r, :, 0:1]
            s_peer = stats_recv[r, :, 128:129]
            m_g = jnp.maximum(m_loc, m_peer)
            s_g = s_loc * jnp.exp(m_loc - m_g) + s_peer * jnp.exp(m_peer - m_g)
            f_mine = jnp.exp(m_loc - m_g) / s_g
            f_peer = jnp.exp(m_peer - m_g) / s_g
            half0 = jnp.where(my_y == 0, f_mine, f_peer)
            half1 = jnp.where(my_y == 0, f_peer, f_mine)
            factors_ref[pl.ds(r * R, R), :] = jnp.concatenate(
                [jnp.broadcast_to(half0, (R, 128)),
                 jnp.broadcast_to(half1, (R, 128))], axis=1)

        for r in range(N_CHUNKS):
            stats_rdma(r).wait_send()
            local_copy(r).wait()
        for k in range(N_HALF):
            y_rdma(k).wait_send()
            f = fwd_rdma(k)
            f.wait_send()
            f.wait_recv()

    return pl.pallas_call(
        body,
        in_specs=[
            pl.BlockSpec(memory_space=pltpu.MemorySpace.VMEM),
            pl.BlockSpec(memory_space=pltpu.MemorySpace.VMEM),
        ],
        out_specs=(
            pl.BlockSpec(memory_space=pl.ANY),
            pl.BlockSpec(memory_space=pltpu.MemorySpace.VMEM),
        ),
        out_shape=(
            jax.ShapeDtypeStruct((T, V), jnp.bfloat16),
            jax.ShapeDtypeStruct((T, 256), jnp.float32),
        ),
        scratch_shapes=[
            pltpu.VMEM((N_CHUNKS, R, Vloc), jnp.bfloat16),
            pltpu.VMEM((N_CHUNKS, R, 256), jnp.float32),
            pltpu.VMEM((N_CHUNKS, R, 256), jnp.float32),
            pltpu.SemaphoreType.DMA((N_CHUNKS,)),
            pltpu.SemaphoreType.DMA((N_CHUNKS,)),
            pltpu.SemaphoreType.DMA((N_HALF,)),
            pltpu.SemaphoreType.DMA((N_HALF,)),
            pltpu.SemaphoreType.DMA((N_HALF,)),
            pltpu.SemaphoreType.DMA((N_HALF,)),
            pltpu.SemaphoreType.DMA((N_CHUNKS,)),
        ],
        compiler_params=pltpu.CompilerParams(
            collective_id=0,
            vmem_limit_bytes=62 * 1024 * 1024,
        ),
    )(x, W)


def _normalize(e_all, factors):
    T, V = e_all.shape
    H = V // 2
    ROWS = 128

    def body(e_ref, f_ref, o_ref):
        f0 = f_ref[:, 0:1]
        f1 = f_ref[:, 128:129]
        o_ref[:, :H] = e_ref[:, :H].astype(jnp.float32) * f0
        o_ref[:, H:] = e_ref[:, H:].astype(jnp.float32) * f1

    return pl.pallas_call(
        body,
        grid=(T // ROWS,),
        in_specs=[
            pl.BlockSpec((ROWS, V), lambda i: (i, 0)),
            pl.BlockSpec((ROWS, 256), lambda i: (i, 0)),
        ],
        out_specs=pl.BlockSpec((ROWS, V), lambda i: (i, 0)),
        out_shape=jax.ShapeDtypeStruct((T, V), jnp.float32),
        compiler_params=pltpu.CompilerParams(
            vmem_limit_bytes=62 * 1024 * 1024,
        ),
    )(e_all, factors)


def kernel(x, W):
    e_all, factors = _gemm_exchange(x, W)
    return _normalize(e_all, factors)


# device time: 89940 ns/iter; 2.6950x vs baseline; 1.0646x over previous
import jax
import jax.numpy as jnp
from jax import lax
from jax.experimental import pallas as pl
from jax.experimental.pallas import tpu as pltpu

N_CHUNKS = 16
N_OWN = N_CHUNKS // 4
N_GBLK = 4


def _gemm_exchange(x, W):
    T, D = x.shape
    _, Vloc = W.shape
    V = 2 * Vloc
    R = T // N_CHUNKS

    def body(x_ref, w_ref, e_all_ref, factors_ref,
             e_buf, stats_send, stats_recv,
             stats_send_sems, stats_recv_sems,
             y_send_sems, y_recv_sems,
             xdir_send_sems, xdir_recv_sems,
             zdir_send_sems, zdir_recv_sems,
             ztr_send_sems, ztr_recv_sems,
             xtr_send_sems, xtr_recv_sems,
             copy_sems):
        my_x = lax.axis_index("x")
        my_y = lax.axis_index("y")
        my_z = lax.axis_index("z")
        zbit = lax.rem(my_z, 2)
        zp = my_z + 1 - 2 * zbit
        y_peer = (my_x, 1 - my_y, my_z)
        x_peer = (1 - my_x, my_y, my_z)
        z_peer = (my_x, my_y, zp)
        col0 = my_y * Vloc
        pcol0 = (1 - my_y) * Vloc

        def idx_own(k):   return 4 * k + 2 * zbit + my_x
        def idx_xnbr(k):  return 4 * k + 2 * zbit + (1 - my_x)
        def idx_znbr(k):  return 4 * k + 2 * (1 - zbit) + my_x

        barrier_sem = pltpu.get_barrier_semaphore()
        for nbr in (y_peer, x_peer, z_peer):
            pl.semaphore_signal(
                barrier_sem, inc=1,
                device_id=nbr, device_id_type=pl.DeviceIdType.MESH,
            )
        pl.semaphore_wait(barrier_sem, 3)

        def stats_rdma(r):
            return pltpu.make_async_remote_copy(
                src_ref=stats_send.at[r],
                dst_ref=stats_recv.at[r],
                send_sem=stats_send_sems.at[r],
                recv_sem=stats_recv_sems.at[r],
                device_id=y_peer,
                device_id_type=pl.DeviceIdType.MESH,
            )

        def y_rdma(k):
            idx = idx_own(k)
            return pltpu.make_async_remote_copy(
                src_ref=e_buf.at[idx],
                dst_ref=e_all_ref.at[pl.ds(idx * R, R), pl.ds(col0, Vloc)],
                send_sem=y_send_sems.at[k],
                recv_sem=y_recv_sems.at[k],
                device_id=y_peer,
                device_id_type=pl.DeviceIdType.MESH,
            )

        def peer_region_rdma(idx, dev, send_sem, recv_sem):
            region = (pl.ds(idx * R, R), pl.ds(pcol0, Vloc))
            return pltpu.make_async_remote_copy(
                src_ref=e_all_ref.at[region[0], region[1]],
                dst_ref=e_all_ref.at[region[0], region[1]],
                send_sem=send_sem, recv_sem=recv_sem,
                device_id=dev, device_id_type=pl.DeviceIdType.MESH,
            )

        def xdir(k):
            return peer_region_rdma(
                idx_own(k), x_peer, xdir_send_sems.at[k], xdir_recv_sems.at[k])

        def zdir(k):
            return peer_region_rdma(
                idx_own(k), z_peer, zdir_send_sems.at[k], zdir_recv_sems.at[k])

        def ztrans(k):
            return peer_region_rdma(
                idx_xnbr(k), z_peer, ztr_send_sems.at[k], ztr_recv_sems.at[k])

        def xtrans(k):
            return peer_region_rdma(
                idx_znbr(k), x_peer,
                xtr_send_sems.at[k - 2], xtr_recv_sems.at[k - 2])

        def direct(k):
            y_rdma(k).wait_recv()
            xdir(k).start()
            zdir(k).start()

        CPG = N_CHUNKS // N_GBLK
        GR = T // N_GBLK
        for g in range(N_GBLK):
            l = jnp.dot(
                x_ref[pl.ds(g * GR, GR), :], w_ref[:, :],
                preferred_element_type=jnp.float32,
            )
            m_loc = jnp.max(l, axis=-1, keepdims=True)
            e = jnp.exp(l - m_loc).astype(jnp.bfloat16)
            s_loc = jnp.sum(e.astype(jnp.float32), axis=-1, keepdims=True)
            for j in range(CPG):
                r = g * CPG + j
                e_buf[r] = e[j * R:(j + 1) * R, :]
                stats_send[r] = jnp.concatenate(
                    [jnp.broadcast_to(m_loc[j * R:(j + 1) * R], (R, 128)),
                     jnp.broadcast_to(s_loc[j * R:(j + 1) * R], (R, 128))],
                    axis=1)
                stats_rdma(r).start()
                local_cp = pltpu.make_async_copy(
                    e_buf.at[r],
                    e_all_ref.at[pl.ds(r * R, R), pl.ds(col0, Vloc)],
                    copy_sems.at[r],
                )
                local_cp.start()
            y_rdma(g).start()
            if g >= 1:
                direct(g - 1)
            if g >= 2 and (g - 2) < 2:
                k = g - 2
                xdir(k).wait_recv()
                ztrans(k).start()
        direct(N_OWN - 1)
        for k in (2, 3):
            zdir(k).wait_recv()
            xtrans(k).start()

        for r in range(N_CHUNKS):
            stats_rdma(r).wait_recv()
            m_loc = stats_send[r, :, 0:1]
            s_loc = stats_send[r, :, 128:129]
            m_peer = stats_recv[r, :, 0:1]
            s_peer = stats_recv[r, :, 128:129]
            m_g = jnp.maximum(m_loc, m_peer)
            s_g = s_loc * jnp.exp(m_loc - m_g) + s_peer * jnp.exp(m_peer - m_g)
            f_mine = jnp.exp(m_loc - m_g) / s_g
            f_peer = jnp.exp(m_peer - m_g) / s_g
            half0 = jnp.where(my_y == 0, f_mine, f_peer)
            half1 = jnp.where(my_y == 0, f_peer, f_mine)
            factors_ref[pl.ds(r * R, R), :] = jnp.concatenate(
                [jnp.broadcast_to(half0, (R, 128)),
                 jnp.broadcast_to(half1, (R, 128))], axis=1)

        for r in range(N_CHUNKS):
            stats_rdma(r).wait_send()
            pltpu.make_async_copy(
                e_buf.at[r],
                e_all_ref.at[pl.ds(r * R, R), pl.ds(col0, Vloc)],
                copy_sems.at[r],
            ).wait()
        for k in range(N_OWN):
            y_rdma(k).wait_send()
            xdir(k).wait_send()
            zdir(k).wait_send()
        for k in (2, 3):
            xdir(k).wait_recv()
        for k in (0, 1):
            zdir(k).wait_recv()
        for k in (0, 1):
            z = ztrans(k)
            z.wait_send()
            z.wait_recv()
        for k in (2, 3):
            xx = xtrans(k)
            xx.wait_send()
            xx.wait_recv()

    return pl.pallas_call(
        body,
        in_specs=[
            pl.BlockSpec(memory_space=pltpu.MemorySpace.VMEM),
            pl.BlockSpec(memory_space=pltpu.MemorySpace.VMEM),
        ],
        out_specs=(
            pl.BlockSpec(memory_space=pl.ANY),
            pl.BlockSpec(memory_space=pltpu.MemorySpace.VMEM),
        ),
        out_shape=(
            jax.ShapeDtypeStruct((T, V), jnp.bfloat16),
            jax.ShapeDtypeStruct((T, 256), jnp.float32),
        ),
        scratch_shapes=[
            pltpu.VMEM((N_CHUNKS, T // N_CHUNKS, Vloc), jnp.bfloat16),
            pltpu.VMEM((N_CHUNKS, T // N_CHUNKS, 256), jnp.float32),
            pltpu.VMEM((N_CHUNKS, T // N_CHUNKS, 256), jnp.float32),
            pltpu.SemaphoreType.DMA((N_CHUNKS,)),
            pltpu.SemaphoreType.DMA((N_CHUNKS,)),
            pltpu.SemaphoreType.DMA((N_OWN,)),
            pltpu.SemaphoreType.DMA((N_OWN,)),
            pltpu.SemaphoreType.DMA((N_OWN,)),
            pltpu.SemaphoreType.DMA((N_OWN,)),
            pltpu.SemaphoreType.DMA((N_OWN,)),
            pltpu.SemaphoreType.DMA((N_OWN,)),
            pltpu.SemaphoreType.DMA((2,)),
            pltpu.SemaphoreType.DMA((2,)),
            pltpu.SemaphoreType.DMA((2,)),
            pltpu.SemaphoreType.DMA((2,)),
            pltpu.SemaphoreType.DMA((N_CHUNKS,)),
        ],
        compiler_params=pltpu.CompilerParams(
            collective_id=0,
            vmem_limit_bytes=62 * 1024 * 1024,
        ),
    )(x, W)


def _normalize(e_all, factors):
    T, V = e_all.shape
    H = V // 2
    ROWS = 128

    def body(e_ref, f_ref, o_ref):
        f0 = f_ref[:, 0:1]
        f1 = f_ref[:, 128:129]
        o_ref[:, :H] = e_ref[:, :H].astype(jnp.float32) * f0
        o_ref[:, H:] = e_ref[:, H:].astype(jnp.float32) * f1

    return pl.pallas_call(
        body,
        grid=(T // ROWS,),
        in_specs=[
            pl.BlockSpec((ROWS, V), lambda i: (i, 0)),
            pl.BlockSpec((ROWS, 256), lambda i: (i, 0)),
        ],
        out_specs=pl.BlockSpec((ROWS, V), lambda i: (i, 0)),
        out_shape=jax.ShapeDtypeStruct((T, V), jnp.float32),
        compiler_params=pltpu.CompilerParams(
            vmem_limit_bytes=62 * 1024 * 1024,
        ),
    )(e_all, factors)


def kernel(x, W):
    e_all, factors = _gemm_exchange(x, W)
    return _normalize(e_all, factors)


# device time: 89480 ns/iter; 2.7089x vs baseline; 1.0051x over previous
import jax
import jax.numpy as jnp
from jax import lax
from jax.experimental import pallas as pl
from jax.experimental.pallas import tpu as pltpu

N_CHUNKS = 16
N_OWN = N_CHUNKS // 4
N_GBLK = 4


def _gemm_exchange(x, W):
    T, D = x.shape
    _, Vloc = W.shape
    V = 2 * Vloc
    R = T // N_CHUNKS

    def body(x_ref, w_ref, e_all_ref, factors_ref,
             e_buf, stats_send, stats_recv,
             stats_send_sems, stats_recv_sems,
             y_send_sems, y_recv_sems,
             xdir_send_sems, xdir_recv_sems,
             zdir_send_sems, zdir_recv_sems,
             ztr_send_sems, ztr_recv_sems,
             xtr_send_sems, xtr_recv_sems,
             copy_sems):
        my_x = lax.axis_index("x")
        my_y = lax.axis_index("y")
        my_z = lax.axis_index("z")
        zbit = lax.rem(my_z, 2)
        zp = my_z + 1 - 2 * zbit
        y_peer = (my_x, 1 - my_y, my_z)
        x_peer = (1 - my_x, my_y, my_z)
        z_peer = (my_x, my_y, zp)
        col0 = my_y * Vloc
        pcol0 = (1 - my_y) * Vloc

        def idx_own(k):   return 4 * k + 2 * zbit + my_x
        def idx_xnbr(k):  return 4 * k + 2 * zbit + (1 - my_x)
        def idx_znbr(k):  return 4 * k + 2 * (1 - zbit) + my_x

        barrier_sem = pltpu.get_barrier_semaphore()
        for nbr in (y_peer, x_peer, z_peer):
            pl.semaphore_signal(
                barrier_sem, inc=1,
                device_id=nbr, device_id_type=pl.DeviceIdType.MESH,
            )
        pl.semaphore_wait(barrier_sem, 3)

        def stats_rdma(r):
            return pltpu.make_async_remote_copy(
                src_ref=stats_send.at[r],
                dst_ref=stats_recv.at[r],
                send_sem=stats_send_sems.at[r],
                recv_sem=stats_recv_sems.at[r],
                device_id=y_peer,
                device_id_type=pl.DeviceIdType.MESH,
            )

        def y_rdma(k):
            idx = idx_own(k)
            return pltpu.make_async_remote_copy(
                src_ref=e_buf.at[idx],
                dst_ref=e_all_ref.at[pl.ds(idx * R, R), pl.ds(col0, Vloc)],
                send_sem=y_send_sems.at[k],
                recv_sem=y_recv_sems.at[k],
                device_id=y_peer,
                device_id_type=pl.DeviceIdType.MESH,
            )

        def peer_region_rdma(idx, dev, send_sem, recv_sem):
            region = (pl.ds(idx * R, R), pl.ds(pcol0, Vloc))
            return pltpu.make_async_remote_copy(
                src_ref=e_all_ref.at[region[0], region[1]],
                dst_ref=e_all_ref.at[region[0], region[1]],
                send_sem=send_sem, recv_sem=recv_sem,
                device_id=dev, device_id_type=pl.DeviceIdType.MESH,
            )

        def xdir(k):
            return peer_region_rdma(
                idx_own(k), x_peer, xdir_send_sems.at[k], xdir_recv_sems.at[k])

        def zdir(k):
            return peer_region_rdma(
                idx_own(k), z_peer, zdir_send_sems.at[k], zdir_recv_sems.at[k])

        def ztrans(k):
            return peer_region_rdma(
                idx_xnbr(k), z_peer, ztr_send_sems.at[k], ztr_recv_sems.at[k])

        def xtrans(k):
            return peer_region_rdma(
                idx_znbr(k), x_peer,
                xtr_send_sems.at[k - 2], xtr_recv_sems.at[k - 2])

        def direct(k):
            y_rdma(k).wait_recv()
            xdir(k).start()
            zdir(k).start()

        CPG = N_CHUNKS // N_GBLK
        GR = T // N_GBLK
        for g in range(N_GBLK):
            l = jnp.dot(
                x_ref[pl.ds(g * GR, GR), :], w_ref[:, :],
                preferred_element_type=jnp.float32,
            )
            m_loc = jnp.max(l, axis=-1, keepdims=True)
            e_f32 = jnp.exp(l - m_loc)
            s_loc = jnp.sum(e_f32, axis=-1, keepdims=True)
            e = e_f32.astype(jnp.bfloat16)
            for j in range(CPG):
                r = g * CPG + j
                e_buf[r] = e[j * R:(j + 1) * R, :]
                stats_send[r] = jnp.concatenate(
                    [jnp.broadcast_to(m_loc[j * R:(j + 1) * R], (R, 128)),
                     jnp.broadcast_to(s_loc[j * R:(j + 1) * R], (R, 128))],
                    axis=1)
                stats_rdma(r).start()
                local_cp = pltpu.make_async_copy(
                    e_buf.at[r],
                    e_all_ref.at[pl.ds(r * R, R), pl.ds(col0, Vloc)],
                    copy_sems.at[r],
                )
                local_cp.start()
            y_rdma(g).start()
            if g >= 1:
                direct(g - 1)
            if g >= 2 and (g - 2) < 2:
                k = g - 2
                xdir(k).wait_recv()
                ztrans(k).start()
        direct(N_OWN - 1)
        for k in (2, 3):
            zdir(k).wait_recv()
            xtrans(k).start()

        for r in range(N_CHUNKS):
            stats_rdma(r).wait_recv()
        m_loc = stats_send[:, :, 0:1]
        s_loc = stats_send[:, :, 128:129]
        m_peer = stats_recv[:, :, 0:1]
        s_peer = stats_recv[:, :, 128:129]
        m_g = jnp.maximum(m_loc, m_peer)
        s_g = s_loc * jnp.exp(m_loc - m_g) + s_peer * jnp.exp(m_peer - m_g)
        f_mine = jnp.exp(m_loc - m_g) / s_g
        f_peer = jnp.exp(m_peer - m_g) / s_g
        half0 = jnp.where(my_y == 0, f_mine, f_peer)
        half1 = jnp.where(my_y == 0, f_peer, f_mine)
        factors_ref[:, :] = jnp.reshape(
            jnp.concatenate(
                [jnp.broadcast_to(half0, (N_CHUNKS, R, 128)),
                 jnp.broadcast_to(half1, (N_CHUNKS, R, 128))], axis=2),
            (T, 256))

        for r in range(N_CHUNKS):
            stats_rdma(r).wait_send()
            pltpu.make_async_copy(
                e_buf.at[r],
                e_all_ref.at[pl.ds(r * R, R), pl.ds(col0, Vloc)],
                copy_sems.at[r],
            ).wait()
        for k in range(N_OWN):
            y_rdma(k).wait_send()
            xdir(k).wait_send()
            zdir(k).wait_send()
        for k in (2, 3):
            xdir(k).wait_recv()
        for k in (0, 1):
            zdir(k).wait_recv()
        for k in (0, 1):
            z = ztrans(k)
            z.wait_send()
            z.wait_recv()
        for k in (2, 3):
            xx = xtrans(k)
            xx.wait_send()
            xx.wait_recv()

    return pl.pallas_call(
        body,
        in_specs=[
            pl.BlockSpec(memory_space=pltpu.MemorySpace.VMEM),
            pl.BlockSpec(memory_space=pltpu.MemorySpace.VMEM),
        ],
        out_specs=(
            pl.BlockSpec(memory_space=pl.ANY),
            pl.BlockSpec(memory_space=pltpu.MemorySpace.VMEM),
        ),
        out_shape=(
            jax.ShapeDtypeStruct((T, V), jnp.bfloat16),
            jax.ShapeDtypeStruct((T, 256), jnp.float32),
        ),
        scratch_shapes=[
            pltpu.VMEM((N_CHUNKS, T // N_CHUNKS, Vloc), jnp.bfloat16),
            pltpu.VMEM((N_CHUNKS, T // N_CHUNKS, 256), jnp.float32),
            pltpu.VMEM((N_CHUNKS, T // N_CHUNKS, 256), jnp.float32),
            pltpu.SemaphoreType.DMA((N_CHUNKS,)),
            pltpu.SemaphoreType.DMA((N_CHUNKS,)),
            pltpu.SemaphoreType.DMA((N_OWN,)),
            pltpu.SemaphoreType.DMA((N_OWN,)),
            pltpu.SemaphoreType.DMA((N_OWN,)),
            pltpu.SemaphoreType.DMA((N_OWN,)),
            pltpu.SemaphoreType.DMA((N_OWN,)),
            pltpu.SemaphoreType.DMA((N_OWN,)),
            pltpu.SemaphoreType.DMA((2,)),
            pltpu.SemaphoreType.DMA((2,)),
            pltpu.SemaphoreType.DMA((2,)),
            pltpu.SemaphoreType.DMA((2,)),
            pltpu.SemaphoreType.DMA((N_CHUNKS,)),
        ],
        compiler_params=pltpu.CompilerParams(
            collective_id=0,
            vmem_limit_bytes=62 * 1024 * 1024,
        ),
    )(x, W)


def _normalize(e_all, factors):
    T, V = e_all.shape
    H = V // 2
    ROWS = 128

    def body(e_ref, f_ref, o_ref):
        f0 = f_ref[:, 0:1]
        f1 = f_ref[:, 128:129]
        o_ref[:, :H] = e_ref[:, :H].astype(jnp.float32) * f0
        o_ref[:, H:] = e_ref[:, H:].astype(jnp.float32) * f1

    return pl.pallas_call(
        body,
        grid=(T // ROWS,),
        in_specs=[
            pl.BlockSpec((ROWS, V), lambda i: (i, 0)),
            pl.BlockSpec((ROWS, 256), lambda i: (i, 0)),
        ],
        out_specs=pl.BlockSpec((ROWS, V), lambda i: (i, 0)),
        out_shape=jax.ShapeDtypeStruct((T, V), jnp.float32),
        compiler_params=pltpu.CompilerParams(
            vmem_limit_bytes=62 * 1024 * 1024,
        ),
    )(e_all, factors)


def kernel(x, W):
    e_all, factors = _gemm_exchange(x, W)
    return _normalize(e_all, factors)
